# Initial kernel scaffold; baseline (speedup 1.0000x reference)
import jax
import jax.numpy as jnp
from jax.experimental import pallas as pl


def kernel(input, emb_table, W1, b1):
    raise NotImplementedError("write your pallas kernel here")


# speedup vs baseline: 65.3718x; 65.3718x over previous
import functools

import jax
import jax.numpy as jnp
from jax import lax
from jax.experimental import pallas as pl
from jax.experimental.pallas import tpu as pltpu
from jax.experimental.pallas import tpu_sc as plsc

VOCAB = 1000000
EMB = 128
HID = 64
BATCH = 4096
L = 200

NC = 2
NS = 16
NW = NC * NS
BPW = BATCH // NW
HALF = L // 2
ROWS_PER_W = BPW * 2
LANES = 16
NCOL = EMB // LANES


def _fire(table_hbm, idx_v, buf, sem, s):
    pltpu.async_copy(table_hbm.at[idx_v.at[2 * s]], buf.at[pl.ds(0, HALF)], sem)
    pltpu.async_copy(table_hbm.at[idx_v.at[2 * s + 1]], buf.at[pl.ds(HALF, HALF)], sem)


def _sc_body(table_hbm, idx_hbm, out_hbm, idx_v, buf0, buf1, res_v, sem0, sem1):
    wid = lax.axis_index("s") * NC + lax.axis_index("c")
    rbase = wid * ROWS_PER_W
    sbase = wid * BPW

    pltpu.sync_copy(idx_hbm.at[pl.ds(rbase, ROWS_PER_W)], idx_v)

    _fire(table_hbm, idx_v, buf0, sem0, 0)
    _fire(table_hbm, idx_v, buf1, sem1, 1)

    def _process(s, buf, sem):
        pltpu.make_async_copy(table_hbm.at[pl.ds(0, L)], buf, sem).wait()

        def rbody(r, acc):
            return tuple(acc[c] + buf[r, pl.ds(c * LANES, LANES)]
                         for c in range(NCOL))

        acc = lax.fori_loop(
            0, L, rbody,
            tuple(jnp.zeros((LANES,), jnp.float32) for _ in range(NCOL)))
        for c in range(NCOL):
            res_v[s, pl.ds(c * LANES, LANES)] = acc[c]

    def loop_body(i, carry):
        for b, (buf, sem) in enumerate(((buf0, sem0), (buf1, sem1))):
            s = 2 * i + b
            _process(s, buf, sem)

            @pl.when(s + 2 < BPW)
            def _():
                _fire(table_hbm, idx_v, buf, sem, s + 2)
        return carry

    lax.fori_loop(0, BPW // 2, loop_body, 0)
    pltpu.sync_copy(res_v, out_hbm.at[pl.ds(sbase, BPW)])


@jax.jit
def _sc_pool(emb_table, idx):
    mesh = plsc.VectorSubcoreMesh(core_axis_name="c", subcore_axis_name="s")
    f = pl.kernel(
        _sc_body,
        out_type=jax.ShapeDtypeStruct((BATCH, EMB), jnp.float32),
        mesh=mesh,
        scratch_types=[
            pltpu.VMEM((ROWS_PER_W, HALF), jnp.int32),
            pltpu.VMEM((L, EMB), jnp.float32),
            pltpu.VMEM((L, EMB), jnp.float32),
            pltpu.VMEM((BPW, EMB), jnp.float32),
            pltpu.SemaphoreType.DMA,
            pltpu.SemaphoreType.DMA,
        ],
    )
    return f(emb_table, idx)


def _tc_body(x_ref, w_ref, b_ref, o_ref):
    o_ref[...] = (
        jnp.dot(x_ref[...] * (1.0 / L), w_ref[...],
                preferred_element_type=jnp.float32)
        + b_ref[...])


@jax.jit
def _tc_fc(pooled, W1, b1):
    bm = 512
    return pl.pallas_call(
        _tc_body,
        grid=(BATCH // bm,),
        in_specs=[
            pl.BlockSpec((bm, EMB), lambda i: (i, 0)),
            pl.BlockSpec((EMB, HID), lambda i: (0, 0)),
            pl.BlockSpec((1, HID), lambda i: (0, 0)),
        ],
        out_specs=pl.BlockSpec((bm, HID), lambda i: (i, 0)),
        out_shape=jax.ShapeDtypeStruct((BATCH, HID), jnp.float32),
    )(pooled, W1, b1)


def kernel(input, emb_table, W1, b1):
    idx = input.astype(jnp.int32).reshape(BATCH * L // HALF, HALF)
    pooled = _sc_pool(emb_table, idx)
    return _tc_fc(pooled, W1, b1.reshape(1, HID))


# speedup vs baseline: 79.8055x; 1.2208x over previous
import functools

import jax
import jax.numpy as jnp
from jax import lax
from jax.experimental import pallas as pl
from jax.experimental.pallas import tpu as pltpu
from jax.experimental.pallas import tpu_sc as plsc

VOCAB = 1000000
EMB = 128
HID = 64
BATCH = 4096
L = 200

NC = 2
NS = 16
NW = NC * NS
BPW = BATCH // NW
LANES = 16
NCOL = EMB // LANES
FIRE_CHUNK = 8


def _sc_body(table_hbm, idxt_hbm, out_hbm, idx_v, acc_v, sem):
    wid = lax.axis_index("s") * NC + lax.axis_index("c")
    sbase = wid * BPW

    idx_cp = pltpu.make_async_copy(
        idxt_hbm.at[:, pl.ds(sbase, BPW)], idx_v, sem)
    idx_cp.start()

    zeros = jnp.zeros((LANES,), jnp.float32)

    def zbody(r, carry):
        for c in range(NCOL):
            acc_v[r, pl.ds(c * LANES, LANES)] = zeros
        return carry

    lax.fori_loop(0, BPW, zbody, 0)
    idx_cp.wait()

    def fire_body(i, carry):
        for j in range(FIRE_CHUNK):
            pltpu.async_copy(
                table_hbm.at[idx_v.at[i * FIRE_CHUNK + j]], acc_v, sem,
                add=True)
        return carry

    lax.fori_loop(0, L // FIRE_CHUNK, fire_body, 0)

    def drain_body(i, carry):
        pltpu.make_async_copy(table_hbm.at[pl.ds(0, BPW)], acc_v, sem).wait()
        return carry

    lax.fori_loop(0, L, drain_body, 0)

    pltpu.sync_copy(acc_v, out_hbm.at[pl.ds(sbase, BPW)])


@jax.jit
def _sc_pool(emb_table, idxt):
    mesh = plsc.VectorSubcoreMesh(core_axis_name="c", subcore_axis_name="s")
    f = pl.kernel(
        _sc_body,
        out_type=jax.ShapeDtypeStruct((BATCH, EMB), jnp.float32),
        mesh=mesh,
        scratch_types=[
            pltpu.VMEM((L, BPW), jnp.int32),
            pltpu.VMEM((BPW, EMB), jnp.float32),
            pltpu.SemaphoreType.DMA,
        ],
    )
    return f(emb_table, idxt)


def _tc_body(x_ref, w_ref, b_ref, o_ref):
    o_ref[...] = (
        jnp.dot(x_ref[...] * (1.0 / L), w_ref[...],
                preferred_element_type=jnp.float32)
        + b_ref[...])


@jax.jit
def _tc_fc(pooled, W1, b1):
    bm = 512
    return pl.pallas_call(
        _tc_body,
        grid=(BATCH // bm,),
        in_specs=[
            pl.BlockSpec((bm, EMB), lambda i: (i, 0)),
            pl.BlockSpec((EMB, HID), lambda i: (0, 0)),
            pl.BlockSpec((1, HID), lambda i: (0, 0)),
        ],
        out_specs=pl.BlockSpec((bm, HID), lambda i: (i, 0)),
        out_shape=jax.ShapeDtypeStruct((BATCH, HID), jnp.float32),
    )(pooled, W1, b1)


def kernel(input, emb_table, W1, b1):
    idxt = input.astype(jnp.int32).T
    pooled = _sc_pool(emb_table, idxt)
    return _tc_fc(pooled, W1, b1.reshape(1, HID))
